# Initial kernel scaffold; baseline (speedup 1.0000x reference)
import jax
import jax.numpy as jnp
from jax.experimental import pallas as pl


def kernel(x, embedding):
    raise NotImplementedError("write your pallas kernel here")


# speedup vs baseline: 1.3475x; 1.3475x over previous
import functools

import jax
import jax.numpy as jnp
from jax import lax
from jax.experimental import pallas as pl
from jax.experimental.pallas import tpu as pltpu
from jax.experimental.pallas import tpu_sc as plsc

D = 256
K = 8192
N = 8192
BM = 256
NBLK = N // BM
COMMITMENT_COST = 0.25


def _vq_body(x_ref, emb_ref, idx_ref, enc_ref, hist_ref, embt_ref,
             esq_ref, ebf_ref, d_ref):
    i = pl.program_id(0)

    @pl.when(i == 0)
    def _():
        e = emb_ref[...]
        esq_ref[...] = jnp.sum(e * e, axis=0, keepdims=True)
        ebf_ref[...] = e.astype(jnp.bfloat16)
        hist_ref[...] = jnp.zeros_like(hist_ref)

    xb = x_ref[...]
    xs = jnp.sum(xb * xb, axis=1, keepdims=True)
    mm = lax.dot_general(xb.astype(jnp.bfloat16), ebf_ref[...],
                         (((1,), (0,)), ((), ())),
                         preferred_element_type=jnp.float32)
    d_ref[...] = (xs - 2.0 * mm) + esq_ref[...]
    d = d_ref[...]
    iota = lax.broadcasted_iota(jnp.int32, (BM, K), 1)
    big = jnp.int32(2 ** 30)

    def win_min(lo, hi):
        dw = d[:, lo:hi]
        mv = jnp.min(dw, axis=1, keepdims=True)
        ii = jnp.min(jnp.where(dw == mv, iota[:, lo:hi], big), axis=1)
        return mv[:, 0], ii

    def rnd(a):
        return a.astype(jnp.bfloat16).astype(jnp.float32)

    m1, i1 = win_min(0, 2816)
    m2, i2 = win_min(2816, 5632)
    m3, i3 = win_min(5632, K)
    acc_v = rnd(m1)
    take2 = m2 < acc_v
    acc_i = jnp.where(take2, i2, i1)
    acc_v = rnd(jnp.where(take2, m2, acc_v))
    idx = jnp.where(m3 < acc_v, i3, acc_i)
    idx_ref[...] = idx.reshape(1, 1, BM)
    onehot = (iota == idx[:, None]).astype(jnp.float32)
    enc_ref[...] = onehot
    hist_ref[...] += jnp.sum(onehot, axis=0, keepdims=True)
    embt_ref[...] = emb_ref[:, pl.ds(i * BM, BM)].T


def _vq_call(flat_x, embedding):
    return pl.pallas_call(
        _vq_body,
        grid=(NBLK,),
        in_specs=[
            pl.BlockSpec((BM, D), lambda i: (i, 0)),
            pl.BlockSpec((D, K), lambda i: (0, 0)),
        ],
        out_specs=[
            pl.BlockSpec((1, 1, BM), lambda i: (i, 0, 0)),
            pl.BlockSpec((BM, K), lambda i: (i, 0)),
            pl.BlockSpec((1, K), lambda i: (0, 0)),
            pl.BlockSpec((BM, D), lambda i: (i, 0)),
        ],
        out_shape=[
            jax.ShapeDtypeStruct((NBLK, 1, BM), jnp.int32),
            jax.ShapeDtypeStruct((N, K), jnp.float32),
            jax.ShapeDtypeStruct((1, K), jnp.float32),
            jax.ShapeDtypeStruct((K, D), jnp.float32),
        ],
        scratch_shapes=[
            pltpu.VMEM((1, K), jnp.float32),
            pltpu.VMEM((D, K), jnp.bfloat16),
            pltpu.VMEM((BM, K), jnp.float32),
        ],
    )(flat_x, embedding)


def _gather_q(embt, idx3):
    info = plsc.get_sparse_core_info()
    nc = info.num_cores
    nw = nc * info.num_subcores
    bpw = N // nw
    mesh = plsc.VectorSubcoreMesh(core_axis_name="c", subcore_axis_name="s")

    @functools.partial(
        pl.kernel,
        out_type=jax.ShapeDtypeStruct((N, D), jnp.float32),
        mesh=mesh,
        scratch_types=[
            pltpu.VMEM((2, 128), jnp.int32),
            pltpu.VMEM((bpw, D), jnp.float32),
            pltpu.SemaphoreType.DMA,
        ],
    )
    def k(embt_hbm, idx_hbm, out_hbm, idx_v, rows_v, sem):
        wid = lax.axis_index("s") * nc + lax.axis_index("c")
        base = wid * bpw
        pltpu.sync_copy(idx_hbm.at[wid], idx_v)
        cp0 = pltpu.async_copy(embt_hbm.at[idx_v.at[0]],
                               rows_v.at[pl.ds(0, 128)], sem)
        cp1 = pltpu.async_copy(embt_hbm.at[idx_v.at[1]],
                               rows_v.at[pl.ds(128, 128)], sem)
        cp0.wait()
        cp1.wait()
        pltpu.sync_copy(rows_v, out_hbm.at[pl.ds(base, bpw)])

    return k(embt, idx3)


LOSS_BM = 1024
LOSS_NBLK = N // LOSS_BM


def _finish_body(q_ref, x_ref, hist_ref, loss_ref, perp_ref, qout_ref):
    i = pl.program_id(0)
    q = q_ref[...]
    xb = x_ref[...]
    diff = q - xb
    qout_ref[...] = xb + diff
    s = jnp.sum(diff * diff)

    @pl.when(i == 0)
    def _():
        loss_ref[...] = jnp.zeros_like(loss_ref)

    loss_ref[...] += s

    @pl.when(i == pl.num_programs(0) - 1)
    def _():
        scale = (1.0 + COMMITMENT_COST) / (N * D)
        loss_ref[...] = loss_ref[...] * scale
        p = hist_ref[...] * (1.0 / N)
        ent = jnp.sum(p * jnp.log(p + 1e-10))
        perp_ref[...] = jnp.full_like(perp_ref, jnp.exp(-ent))


def _finish_call(q_flat, flat_x, hist):
    return pl.pallas_call(
        _finish_body,
        grid=(LOSS_NBLK,),
        in_specs=[
            pl.BlockSpec((LOSS_BM, D), lambda i: (i, 0)),
            pl.BlockSpec((LOSS_BM, D), lambda i: (i, 0)),
            pl.BlockSpec((1, K), lambda i: (0, 0)),
        ],
        out_specs=[
            pl.BlockSpec((1, 1), lambda i: (0, 0)),
            pl.BlockSpec((1, 1), lambda i: (0, 0)),
            pl.BlockSpec((LOSS_BM, D), lambda i: (i, 0)),
        ],
        out_shape=[
            jax.ShapeDtypeStruct((1, 1), jnp.float32),
            jax.ShapeDtypeStruct((1, 1), jnp.float32),
            jax.ShapeDtypeStruct((N, D), jnp.float32),
        ],
    )(q_flat, flat_x, hist)


def kernel(x, embedding):
    flat_x = x.reshape(N, D)
    idx3, enc, hist, embt = _vq_call(flat_x, embedding)
    q_flat = _gather_q(embt, idx3.reshape(NBLK, 2, 128))
    loss11, perp11, quant = _finish_call(q_flat, flat_x, hist)
    return (loss11.reshape(()), quant.reshape(x.shape), perp11.reshape(()), enc)


# speedup vs baseline: 1.6821x; 1.2483x over previous
import functools

import jax
import jax.numpy as jnp
from jax import lax
from jax.experimental import pallas as pl
from jax.experimental.pallas import tpu as pltpu
from jax.experimental.pallas import tpu_sc as plsc

D = 256
K = 8192
N = 8192
BM = 256
NBLK = N // BM
COMMITMENT_COST = 0.25


def _vq_body(x_ref, emb_ref, idx_ref, enc_ref, hist_ref, embt_ref,
             esq_ref, ebf_ref, iotaf_ref, hist8_ref, d_ref):
    i = pl.program_id(0)

    @pl.when(i == 0)
    def _():
        e = emb_ref[...]
        esq_ref[...] = jnp.sum(e * e, axis=0, keepdims=True)
        ebf_ref[...] = e.astype(jnp.bfloat16)
        iotaf_ref[...] = lax.broadcasted_iota(jnp.int32, (1, K), 1).astype(jnp.float32)
        hist8_ref[...] = jnp.zeros_like(hist8_ref)

    xb = x_ref[...]
    xs = jnp.sum(xb * xb, axis=1, keepdims=True)
    mm = lax.dot_general(xb.astype(jnp.bfloat16), ebf_ref[...],
                         (((1,), (0,)), ((), ())),
                         preferred_element_type=jnp.float32)
    d_ref[...] = (xs - 2.0 * mm) + esq_ref[...]
    d = d_ref[...]
    iotaf = iotaf_ref[...]
    big = jnp.float32(2.0 ** 30)

    def win_min(lo, hi):
        dw = d[:, lo:hi]
        mv = jnp.min(dw, axis=1, keepdims=True)
        ii = jnp.min(jnp.where(dw == mv, iotaf[:, lo:hi], big), axis=1)
        return mv[:, 0], ii

    def rnd(a):
        return a.astype(jnp.bfloat16).astype(jnp.float32)

    m1, i1 = win_min(0, 2816)
    m2, i2 = win_min(2816, 5632)
    m3, i3 = win_min(5632, K)
    acc_v = rnd(m1)
    take2 = m2 < acc_v
    acc_i = jnp.where(take2, i2, i1)
    acc_v = rnd(jnp.where(take2, m2, acc_v))
    idx = jnp.where(m3 < acc_v, i3, acc_i)
    idx_ref[...] = idx.astype(jnp.int32).reshape(1, 1, BM)
    onehot = jnp.where(iotaf == idx[:, None], 1.0, 0.0)
    enc_ref[...] = onehot
    hist8_ref[...] += lax.dot_general(
        jnp.ones((8, BM), jnp.float32), onehot,
        (((1,), (0,)), ((), ())), preferred_element_type=jnp.float32)
    embt_ref[...] = emb_ref[:, pl.ds(i * BM, BM)].T

    @pl.when(i == NBLK - 1)
    def _():
        hist_ref[...] = jnp.sum(hist8_ref[...], axis=0, keepdims=True) * 0.125


def _vq_call(flat_x, embedding):
    return pl.pallas_call(
        _vq_body,
        grid=(NBLK,),
        in_specs=[
            pl.BlockSpec((BM, D), lambda i: (i, 0)),
            pl.BlockSpec((D, K), lambda i: (0, 0)),
        ],
        out_specs=[
            pl.BlockSpec((1, 1, BM), lambda i: (i, 0, 0)),
            pl.BlockSpec((BM, K), lambda i: (i, 0)),
            pl.BlockSpec((1, K), lambda i: (0, 0)),
            pl.BlockSpec((BM, D), lambda i: (i, 0)),
        ],
        out_shape=[
            jax.ShapeDtypeStruct((NBLK, 1, BM), jnp.int32),
            jax.ShapeDtypeStruct((N, K), jnp.float32),
            jax.ShapeDtypeStruct((1, K), jnp.float32),
            jax.ShapeDtypeStruct((K, D), jnp.float32),
        ],
        scratch_shapes=[
            pltpu.VMEM((1, K), jnp.float32),
            pltpu.VMEM((D, K), jnp.bfloat16),
            pltpu.VMEM((1, K), jnp.float32),
            pltpu.VMEM((8, K), jnp.float32),
            pltpu.VMEM((BM, K), jnp.float32),
        ],
    )(flat_x, embedding)


def _gather_q(embt, idx3):
    info = plsc.get_sparse_core_info()
    nc = info.num_cores
    nw = nc * info.num_subcores
    bpw = N // nw
    mesh = plsc.VectorSubcoreMesh(core_axis_name="c", subcore_axis_name="s")

    @functools.partial(
        pl.kernel,
        out_type=jax.ShapeDtypeStruct((N, D), jnp.float32),
        mesh=mesh,
        scratch_types=[
            pltpu.VMEM((2, 128), jnp.int32),
            pltpu.VMEM((bpw, D), jnp.float32),
            pltpu.SemaphoreType.DMA,
        ],
    )
    def k(embt_hbm, idx_hbm, out_hbm, idx_v, rows_v, sem):
        wid = lax.axis_index("s") * nc + lax.axis_index("c")
        base = wid * bpw
        pltpu.sync_copy(idx_hbm.at[wid], idx_v)
        cp0 = pltpu.async_copy(embt_hbm.at[idx_v.at[0]],
                               rows_v.at[pl.ds(0, 128)], sem)
        cp1 = pltpu.async_copy(embt_hbm.at[idx_v.at[1]],
                               rows_v.at[pl.ds(128, 128)], sem)
        cp0.wait()
        cp1.wait()
        pltpu.sync_copy(rows_v, out_hbm.at[pl.ds(base, bpw)])

    return k(embt, idx3)


LOSS_BM = 1024
LOSS_NBLK = N // LOSS_BM


def _finish_body(q_ref, x_ref, hist_ref, loss_ref, perp_ref, qout_ref):
    i = pl.program_id(0)
    q = q_ref[...]
    xb = x_ref[...]
    diff = q - xb
    qout_ref[...] = xb + diff
    s = jnp.sum(diff * diff)

    @pl.when(i == 0)
    def _():
        loss_ref[...] = jnp.zeros_like(loss_ref)

    loss_ref[...] += s

    @pl.when(i == pl.num_programs(0) - 1)
    def _():
        scale = (1.0 + COMMITMENT_COST) / (N * D)
        loss_ref[...] = loss_ref[...] * scale
        p = hist_ref[...] * (1.0 / N)
        ent = jnp.sum(p * jnp.log(p + 1e-10))
        perp_ref[...] = jnp.full_like(perp_ref, jnp.exp(-ent))


def _finish_call(q_flat, flat_x, hist):
    return pl.pallas_call(
        _finish_body,
        grid=(LOSS_NBLK,),
        in_specs=[
            pl.BlockSpec((LOSS_BM, D), lambda i: (i, 0)),
            pl.BlockSpec((LOSS_BM, D), lambda i: (i, 0)),
            pl.BlockSpec((1, K), lambda i: (0, 0)),
        ],
        out_specs=[
            pl.BlockSpec((1, 1), lambda i: (0, 0)),
            pl.BlockSpec((1, 1), lambda i: (0, 0)),
            pl.BlockSpec((LOSS_BM, D), lambda i: (i, 0)),
        ],
        out_shape=[
            jax.ShapeDtypeStruct((1, 1), jnp.float32),
            jax.ShapeDtypeStruct((1, 1), jnp.float32),
            jax.ShapeDtypeStruct((N, D), jnp.float32),
        ],
    )(q_flat, flat_x, hist)


def kernel(x, embedding):
    flat_x = x.reshape(N, D)
    idx3, enc, hist, embt = _vq_call(flat_x, embedding)
    q_flat = _gather_q(embt, idx3.reshape(NBLK, 2, 128))
    loss11, perp11, quant = _finish_call(q_flat, flat_x, hist)
    return (loss11.reshape(()), quant.reshape(x.shape), perp11.reshape(()), enc)


# speedup vs baseline: 1.8046x; 1.0728x over previous
import functools

import jax
import jax.numpy as jnp
from jax import lax
from jax.experimental import pallas as pl
from jax.experimental.pallas import tpu as pltpu
from jax.experimental.pallas import tpu_sc as plsc

D = 256
K = 8192
N = 8192
BM = 256
NBLK = N // BM
COMMITMENT_COST = 0.25


def _vq_body(x_ref, emb_ref, idx_ref, enc_ref, loss_ref, perp_ref, embt_ref,
             esq_ref, ebf_ref, iotaf_ref, hist8_ref):
    i = pl.program_id(0)

    @pl.when(i == 0)
    def _():
        e = emb_ref[...]
        esq_ref[...] = jnp.sum(e * e, axis=0, keepdims=True)
        ebf_ref[...] = e.astype(jnp.bfloat16)
        iotaf_ref[...] = lax.broadcasted_iota(jnp.int32, (1, K), 1).astype(jnp.float32)
        hist8_ref[...] = jnp.zeros_like(hist8_ref)
        loss_ref[...] = jnp.zeros_like(loss_ref)

    xb = x_ref[...]
    xs = jnp.sum(xb * xb, axis=1, keepdims=True)
    mm = lax.dot_general(xb.astype(jnp.bfloat16), ebf_ref[...],
                         (((1,), (0,)), ((), ())),
                         preferred_element_type=jnp.float32)
    d = (xs - 2.0 * mm) + esq_ref[...]
    iotaf = iotaf_ref[...]
    big = jnp.float32(2.0 ** 30)

    def win_min(lo, hi):
        dw = d[:, lo:hi]
        mv = jnp.min(dw, axis=1, keepdims=True)
        ii = jnp.min(jnp.where(dw == mv, iotaf[:, lo:hi], big), axis=1)
        return mv[:, 0], ii

    def rnd(a):
        return a.astype(jnp.bfloat16).astype(jnp.float32)

    m1, i1 = win_min(0, 2816)
    m2, i2 = win_min(2816, 5632)
    m3, i3 = win_min(5632, K)
    acc_v = rnd(m1)
    take2 = m2 < acc_v
    acc_i = jnp.where(take2, i2, i1)
    acc_m = jnp.where(take2, m2, m1)
    acc_v = rnd(jnp.where(take2, m2, acc_v))
    take3 = m3 < acc_v
    idx = jnp.where(take3, i3, acc_i)
    dsel = jnp.where(take3, m3, acc_m)
    idx_ref[...] = idx.astype(jnp.int32).reshape(1, 1, BM)
    onehot = jnp.where(iotaf == idx[:, None], 1.0, 0.0)
    enc_ref[...] = onehot
    hist8_ref[...] += lax.dot_general(
        jnp.ones((8, BM), jnp.float32), onehot,
        (((1,), (0,)), ((), ())), preferred_element_type=jnp.float32)
    embt_ref[...] = emb_ref[:, pl.ds(i * BM, BM)].T
    loss_ref[...] += jnp.sum(dsel)

    @pl.when(i == NBLK - 1)
    def _():
        loss_ref[...] = loss_ref[...] * ((1.0 + COMMITMENT_COST) / (N * D))
        p = jnp.sum(hist8_ref[...], axis=0, keepdims=True) * (0.125 / N)
        ent = jnp.sum(p * jnp.log(p + 1e-10))
        perp_ref[...] = jnp.full_like(perp_ref, jnp.exp(-ent))


def _vq_call(flat_x, embedding):
    return pl.pallas_call(
        _vq_body,
        grid=(NBLK,),
        in_specs=[
            pl.BlockSpec((BM, D), lambda i: (i, 0)),
            pl.BlockSpec((D, K), lambda i: (0, 0)),
        ],
        out_specs=[
            pl.BlockSpec((1, 1, BM), lambda i: (i, 0, 0)),
            pl.BlockSpec((BM, K), lambda i: (i, 0)),
            pl.BlockSpec((1, 1), lambda i: (0, 0)),
            pl.BlockSpec((1, 1), lambda i: (0, 0)),
            pl.BlockSpec((BM, D), lambda i: (i, 0)),
        ],
        out_shape=[
            jax.ShapeDtypeStruct((NBLK, 1, BM), jnp.int32),
            jax.ShapeDtypeStruct((N, K), jnp.float32),
            jax.ShapeDtypeStruct((1, 1), jnp.float32),
            jax.ShapeDtypeStruct((1, 1), jnp.float32),
            jax.ShapeDtypeStruct((K, D), jnp.float32),
        ],
        scratch_shapes=[
            pltpu.VMEM((1, K), jnp.float32),
            pltpu.VMEM((D, K), jnp.bfloat16),
            pltpu.VMEM((1, K), jnp.float32),
            pltpu.VMEM((8, K), jnp.float32),
        ],
    )(flat_x, embedding)


def _gather_q(embt, idx3):
    info = plsc.get_sparse_core_info()
    nc = info.num_cores
    nw = nc * info.num_subcores
    bpw = N // nw
    mesh = plsc.VectorSubcoreMesh(core_axis_name="c", subcore_axis_name="s")

    @functools.partial(
        pl.kernel,
        out_type=jax.ShapeDtypeStruct((N, D), jnp.float32),
        mesh=mesh,
        scratch_types=[
            pltpu.VMEM((2, 128), jnp.int32),
            pltpu.VMEM((bpw, D), jnp.float32),
            pltpu.SemaphoreType.DMA,
        ],
    )
    def k(embt_hbm, idx_hbm, out_hbm, idx_v, rows_v, sem):
        wid = lax.axis_index("s") * nc + lax.axis_index("c")
        base = wid * bpw
        pltpu.sync_copy(idx_hbm.at[wid], idx_v)
        cp0 = pltpu.async_copy(embt_hbm.at[idx_v.at[0]],
                               rows_v.at[pl.ds(0, 128)], sem)
        cp1 = pltpu.async_copy(embt_hbm.at[idx_v.at[1]],
                               rows_v.at[pl.ds(128, 128)], sem)
        cp0.wait()
        cp1.wait()
        pltpu.sync_copy(rows_v, out_hbm.at[pl.ds(base, bpw)])

    return k(embt, idx3)


def kernel(x, embedding):
    flat_x = x.reshape(N, D)
    idx3, enc, loss11, perp11, embt = _vq_call(flat_x, embedding)
    q_flat = _gather_q(embt, idx3.reshape(NBLK, 2, 128))
    return (loss11.reshape(()), q_flat.reshape(x.shape), perp11.reshape(()), enc)
